# baseline (speedup 1.0000x reference)
import functools
import math

import jax
import jax.numpy as jnp
from jax import lax
from jax.experimental import pallas as pl
from jax.experimental.pallas import tpu as pltpu
from jax.experimental.pallas import tpu_sc as plsc

BLK1 = 1024
BLK2 = 512
KA = 9
KPAD = 16


def _bf16rn(a):
    u = lax.bitcast_convert_type(a, jnp.uint32)
    r = (u >> 16) & jnp.uint32(1)
    u2 = (u + jnp.uint32(0x7FFF) + r) & jnp.uint32(0xFFFF0000)
    return lax.bitcast_convert_type(u2, jnp.float32)


def _topk_qkv_body(cr_ref, cc_ref, x_ref, w_ref, q_ref, kv_ref, idx_ref):
    b = pl.program_id(0)
    n_total = cc_ref.shape[2]
    qkv = jnp.dot(x_ref[0], w_ref[...], preferred_element_type=jnp.float32)
    d_model = x_ref.shape[2]
    q_ref[0] = qkv[:, :d_model]
    kv_ref[0] = qkv[:, d_model:]

    crow = cr_ref[0]
    cxr = crow[:, 0:1]
    cyr = crow[:, 1:2]
    cx = cc_ref[0, 0:1, :]
    cy = cc_ref[0, 1:2, :]
    sqr = cxr * cxr + cyr * cyr
    sqc = cx * cx + cy * cy
    cb_r = _bf16rn(crow).astype(jnp.bfloat16)
    cb_c = _bf16rn(cc_ref[0]).astype(jnp.bfloat16)
    prod = jnp.dot(cb_r, cb_c, preferred_element_type=jnp.float32)
    d2 = (sqr + sqc) - 2.0 * prod
    dist = jnp.maximum(d2, jnp.float32(1e-12))

    colid = lax.broadcasted_iota(jnp.int32, dist.shape, 1).astype(jnp.float32)
    lane16 = lax.broadcasted_iota(jnp.int32, (dist.shape[0], KPAD), 1)
    acc = jnp.zeros((dist.shape[0], KPAD), jnp.int32)
    nf = jnp.float32(n_total)
    d = dist
    for t in range(KA):
        m = jnp.min(d, axis=1, keepdims=True)
        hit = jnp.where(d == m, colid, nf)
        idxv = jnp.min(hit, axis=1, keepdims=True)
        acc = jnp.where(lane16 == t,
                        idxv.astype(jnp.int32) + b * n_total, acc)
        d = jnp.where(colid == idxv, jnp.float32(jnp.inf), d)
    idx_ref[0] = acc


def _topk_qkv(coords, coords_t, x, w_qkv):
    B, N, D = x.shape
    grid = (B, N // BLK1)
    return pl.pallas_call(
        _topk_qkv_body,
        grid=grid,
        in_specs=[
            pl.BlockSpec((1, BLK1, 2), lambda b, i: (b, i, 0)),
            pl.BlockSpec((1, 2, N), lambda b, i: (b, 0, 0)),
            pl.BlockSpec((1, BLK1, D), lambda b, i: (b, i, 0)),
            pl.BlockSpec((D, 3 * D), lambda b, i: (0, 0)),
        ],
        out_specs=[
            pl.BlockSpec((1, BLK1, D), lambda b, i: (b, i, 0)),
            pl.BlockSpec((1, BLK1, 2 * D), lambda b, i: (b, i, 0)),
            pl.BlockSpec((1, BLK1, KPAD), lambda b, i: (b, i, 0)),
        ],
        out_shape=[
            jax.ShapeDtypeStruct((B, N, D), jnp.float32),
            jax.ShapeDtypeStruct((B, N, 2 * D), jnp.float32),
            jax.ShapeDtypeStruct((B, N, KPAD), jnp.int32),
        ],
        compiler_params=pltpu.CompilerParams(
            dimension_semantics=("parallel", "parallel")),
    )(coords, coords_t, x, w_qkv)


def _sc_gather(kv_flat, idx_flat):
    rows, width = idx_flat.shape[0], kv_flat.shape[1]
    info = plsc.get_sparse_core_info()
    nc, ns = info.num_cores, info.num_subcores
    nw = nc * ns
    rpw = rows // nw
    ch = 128
    nchunk = rpw // ch
    mesh = plsc.VectorSubcoreMesh(core_axis_name="c", subcore_axis_name="s")

    @functools.partial(
        pl.kernel,
        mesh=mesh,
        out_type=jax.ShapeDtypeStruct((rows, width), jnp.float32),
        scratch_types=[
            pltpu.VMEM((ch,), jnp.int32),
            pltpu.VMEM((ch, width), jnp.float32),
            pltpu.SemaphoreType.DMA,
        ],
    )
    def gk(kv_hbm, idx_hbm, out_hbm, idx_v, rows_v, sem):
        wid = lax.axis_index("s") * nc + lax.axis_index("c")
        base = wid * rpw

        def body(i, carry):
            off = base + i * ch
            pltpu.sync_copy(idx_hbm.at[pl.ds(off, ch)], idx_v)
            pltpu.async_copy(kv_hbm.at[idx_v], rows_v, sem).wait()
            pltpu.sync_copy(rows_v, out_hbm.at[pl.ds(off, ch)])
            return carry

        lax.fori_loop(0, nchunk, body, 0)

    return gk(kv_flat, idx_flat)


def _attn_body(q_ref, kvg_ref, x_ref, wp_ref, bp_ref, o_ref):
    blk, d_model = q_ref.shape[1], q_ref.shape[2]
    heads = 4
    hd = d_model // heads
    q = q_ref[0]
    r128 = lax.broadcasted_iota(jnp.int32, (d_model, heads), 0)
    c4 = lax.broadcasted_iota(jnp.int32, (d_model, heads), 1)
    sel = (r128 // hd == c4).astype(jnp.float32)
    r4 = lax.broadcasted_iota(jnp.int32, (heads, d_model), 0)
    c128 = lax.broadcasted_iota(jnp.int32, (heads, d_model), 1)
    selt = (c128 // hd == r4).astype(jnp.float32)

    scale = 1.0 / math.sqrt(hd)
    s = []
    for j in range(KA):
        kj = kvg_ref[j, 0, :, :d_model]
        s.append(jnp.dot(q * kj, sel,
                         preferred_element_type=jnp.float32) * scale)
    m = s[0]
    for j in range(1, KA):
        m = jnp.maximum(m, s[j])
    e = [jnp.exp(sj - m) for sj in s]
    den = e[0]
    for j in range(1, KA):
        den = den + e[j]
    inv = 1.0 / den
    out = jnp.zeros((blk, d_model), jnp.float32)
    for j in range(KA):
        wj = e[j] * inv
        wb = jnp.dot(wj, selt, preferred_element_type=jnp.float32)
        vj = kvg_ref[j, 0, :, d_model:]
        out = out + wb * vj
    g = out * 0.5 * (1.0 + lax.erf(out * (1.0 / math.sqrt(2.0))))
    res = jnp.dot(g, wp_ref[...], preferred_element_type=jnp.float32)
    o_ref[0] = x_ref[0] + res + bp_ref[...]


def _attn(q, kvg, x, w_proj, b_proj2):
    B, N, D = x.shape
    grid = (B, N // BLK2)
    return pl.pallas_call(
        _attn_body,
        grid=grid,
        in_specs=[
            pl.BlockSpec((1, BLK2, D), lambda b, i: (b, i, 0)),
            pl.BlockSpec((KA, 1, BLK2, 2 * D), lambda b, i: (0, b, i, 0)),
            pl.BlockSpec((1, BLK2, D), lambda b, i: (b, i, 0)),
            pl.BlockSpec((D, D), lambda b, i: (0, 0)),
            pl.BlockSpec((1, D), lambda b, i: (0, 0)),
        ],
        out_specs=pl.BlockSpec((1, BLK2, D), lambda b, i: (b, i, 0)),
        out_shape=jax.ShapeDtypeStruct((B, N, D), jnp.float32),
        compiler_params=pltpu.CompilerParams(
            dimension_semantics=("parallel", "parallel")),
    )(q, kvg, x, w_proj, b_proj2)


def kernel(x, coords, W_qkv, W_proj, b_proj):
    B, N, D = x.shape
    coords_t = coords.transpose(0, 2, 1)
    bp2 = b_proj.reshape(1, D)
    outs = []
    for b in range(B):
        xb = x[b:b + 1]
        q, kv, idx16 = _topk_qkv(coords[b:b + 1], coords_t[b:b + 1], xb, W_qkv)
        idxf = jnp.transpose(idx16[:, :, :KA], (2, 0, 1)).reshape(-1)
        kvg = _sc_gather(kv.reshape(N, 2 * D), idxf)
        kvg = kvg.reshape(KA, 1, N, 2 * D)
        outs.append(_attn(q, kvg, xb, W_proj, bp2))
    return jnp.concatenate(outs, axis=0)


# speedup vs baseline: 1.0653x; 1.0653x over previous
import functools
import math

import jax
import jax.numpy as jnp
from jax import lax
from jax.experimental import pallas as pl
from jax.experimental.pallas import tpu as pltpu
from jax.experimental.pallas import tpu_sc as plsc

BLK1 = 256
BLK2 = 512
KA = 9
KPAD = 16


def _bf16rn(a):
    u = lax.bitcast_convert_type(a, jnp.uint32)
    r = (u >> 16) & jnp.uint32(1)
    u2 = (u + jnp.uint32(0x7FFF) + r) & jnp.uint32(0xFFFF0000)
    return lax.bitcast_convert_type(u2, jnp.float32)


def _topk_qkv_body(cr_ref, cc_ref, x_ref, w_ref, q_ref, kv_ref, idx_ref):
    b = pl.program_id(0)
    n_total = cc_ref.shape[2]
    qkv = jnp.dot(x_ref[0], w_ref[...], preferred_element_type=jnp.float32)
    d_model = x_ref.shape[2]
    q_ref[0] = qkv[:, :d_model]
    kv_ref[0] = qkv[:, d_model:]

    crow = cr_ref[0]
    cxr = crow[:, 0:1]
    cyr = crow[:, 1:2]
    cx = cc_ref[0, 0:1, :]
    cy = cc_ref[0, 1:2, :]
    sqr = cxr * cxr + cyr * cyr
    sqc = cx * cx + cy * cy
    cb_r = _bf16rn(crow).astype(jnp.bfloat16)
    cb_c = _bf16rn(cc_ref[0]).astype(jnp.bfloat16)
    prod = jnp.dot(cb_r, cb_c, preferred_element_type=jnp.float32)
    d2 = (sqr + sqc) - 2.0 * prod
    dist = jnp.maximum(d2, jnp.float32(1e-12))

    colid = lax.broadcasted_iota(jnp.int32, dist.shape, 1).astype(jnp.float32)
    lane16 = lax.broadcasted_iota(jnp.int32, (dist.shape[0], KPAD), 1)
    acc = jnp.zeros((dist.shape[0], KPAD), jnp.int32)
    nf = jnp.float32(n_total)
    d = dist
    for t in range(KA):
        m = jnp.min(d, axis=1, keepdims=True)
        hit = jnp.where(d == m, colid, nf)
        idxv = jnp.min(hit, axis=1, keepdims=True)
        acc = jnp.where(lane16 == t,
                        idxv.astype(jnp.int32) + b * n_total, acc)
        d = jnp.where(colid == idxv, jnp.float32(jnp.inf), d)
    idx_ref[0] = acc


def _topk_qkv(coords, coords_t, x, w_qkv):
    B, N, D = x.shape
    grid = (B, N // BLK1)
    return pl.pallas_call(
        _topk_qkv_body,
        grid=grid,
        in_specs=[
            pl.BlockSpec((1, BLK1, 2), lambda b, i: (b, i, 0)),
            pl.BlockSpec((1, 2, N), lambda b, i: (b, 0, 0)),
            pl.BlockSpec((1, BLK1, D), lambda b, i: (b, i, 0)),
            pl.BlockSpec((D, 3 * D), lambda b, i: (0, 0)),
        ],
        out_specs=[
            pl.BlockSpec((1, BLK1, D), lambda b, i: (b, i, 0)),
            pl.BlockSpec((1, BLK1, 2 * D), lambda b, i: (b, i, 0)),
            pl.BlockSpec((1, BLK1, KPAD), lambda b, i: (b, i, 0)),
        ],
        out_shape=[
            jax.ShapeDtypeStruct((B, N, D), jnp.float32),
            jax.ShapeDtypeStruct((B, N, 2 * D), jnp.float32),
            jax.ShapeDtypeStruct((B, N, KPAD), jnp.int32),
        ],
        compiler_params=pltpu.CompilerParams(
            dimension_semantics=("parallel", "parallel")),
    )(coords, coords_t, x, w_qkv)


def _sc_gather(kv_flat, idx_flat):
    rows, width = idx_flat.shape[0], kv_flat.shape[1]
    info = plsc.get_sparse_core_info()
    nc, ns = info.num_cores, info.num_subcores
    nw = nc * ns
    rpw = rows // nw
    ch = 128
    nchunk = rpw // ch
    mesh = plsc.VectorSubcoreMesh(core_axis_name="c", subcore_axis_name="s")

    @functools.partial(
        pl.kernel,
        mesh=mesh,
        out_type=jax.ShapeDtypeStruct((rows, width), jnp.float32),
        scratch_types=[
            pltpu.VMEM((ch,), jnp.int32),
            pltpu.VMEM((ch, width), jnp.float32),
            pltpu.SemaphoreType.DMA,
        ],
    )
    def gk(kv_hbm, idx_hbm, out_hbm, idx_v, rows_v, sem):
        wid = lax.axis_index("s") * nc + lax.axis_index("c")
        base = wid * rpw

        def body(i, carry):
            off = base + i * ch
            pltpu.sync_copy(idx_hbm.at[pl.ds(off, ch)], idx_v)
            pltpu.async_copy(kv_hbm.at[idx_v], rows_v, sem).wait()
            pltpu.sync_copy(rows_v, out_hbm.at[pl.ds(off, ch)])
            return carry

        lax.fori_loop(0, nchunk, body, 0)

    return gk(kv_flat, idx_flat)


def _attn_body(q_ref, kvg_ref, x_ref, wp_ref, bp_ref, o_ref):
    blk, d_model = q_ref.shape[1], q_ref.shape[2]
    heads = 4
    hd = d_model // heads
    q = q_ref[0]
    r128 = lax.broadcasted_iota(jnp.int32, (d_model, heads), 0)
    c4 = lax.broadcasted_iota(jnp.int32, (d_model, heads), 1)
    sel = (r128 // hd == c4).astype(jnp.float32)
    r4 = lax.broadcasted_iota(jnp.int32, (heads, d_model), 0)
    c128 = lax.broadcasted_iota(jnp.int32, (heads, d_model), 1)
    selt = (c128 // hd == r4).astype(jnp.float32)

    scale = 1.0 / math.sqrt(hd)
    s = []
    for j in range(KA):
        kj = kvg_ref[j, 0, :, :d_model]
        s.append(jnp.dot(q * kj, sel,
                         preferred_element_type=jnp.float32) * scale)
    m = s[0]
    for j in range(1, KA):
        m = jnp.maximum(m, s[j])
    e = [jnp.exp(sj - m) for sj in s]
    den = e[0]
    for j in range(1, KA):
        den = den + e[j]
    inv = 1.0 / den
    out = jnp.zeros((blk, d_model), jnp.float32)
    for j in range(KA):
        wj = e[j] * inv
        wb = jnp.dot(wj, selt, preferred_element_type=jnp.float32)
        vj = kvg_ref[j, 0, :, d_model:]
        out = out + wb * vj
    g = out * 0.5 * (1.0 + lax.erf(out * (1.0 / math.sqrt(2.0))))
    res = jnp.dot(g, wp_ref[...], preferred_element_type=jnp.float32)
    o_ref[0] = x_ref[0] + res + bp_ref[...]


def _attn(q, kvg, x, w_proj, b_proj2):
    B, N, D = x.shape
    grid = (B, N // BLK2)
    return pl.pallas_call(
        _attn_body,
        grid=grid,
        in_specs=[
            pl.BlockSpec((1, BLK2, D), lambda b, i: (b, i, 0)),
            pl.BlockSpec((KA, 1, BLK2, 2 * D), lambda b, i: (0, b, i, 0)),
            pl.BlockSpec((1, BLK2, D), lambda b, i: (b, i, 0)),
            pl.BlockSpec((D, D), lambda b, i: (0, 0)),
            pl.BlockSpec((1, D), lambda b, i: (0, 0)),
        ],
        out_specs=pl.BlockSpec((1, BLK2, D), lambda b, i: (b, i, 0)),
        out_shape=jax.ShapeDtypeStruct((B, N, D), jnp.float32),
        compiler_params=pltpu.CompilerParams(
            dimension_semantics=("parallel", "parallel")),
    )(q, kvg, x, w_proj, b_proj2)


def kernel(x, coords, W_qkv, W_proj, b_proj):
    B, N, D = x.shape
    coords_t = coords.transpose(0, 2, 1)
    bp2 = b_proj.reshape(1, D)
    outs = []
    for b in range(B):
        xb = x[b:b + 1]
        q, kv, idx16 = _topk_qkv(coords[b:b + 1], coords_t[b:b + 1], xb, W_qkv)
        idxf = jnp.transpose(idx16[:, :, :KA], (2, 0, 1)).reshape(-1)
        kvg = _sc_gather(kv.reshape(N, 2 * D), idxf)
        kvg = kvg.reshape(KA, 1, N, 2 * D)
        outs.append(_attn(q, kvg, xb, W_proj, bp2))
    return jnp.concatenate(outs, axis=0)
